# baseline (speedup 1.0000x reference)
import jax
import jax.numpy as jnp
from jax import lax
from jax.experimental import pallas as pl
from jax.experimental.pallas import tpu as pltpu
from jax.experimental.pallas import tpu_sc as plsc

B = 16384
F = 26
FIELD_DIM = 38461
NC, NS, L = 2, 16, 16
NW = NC * NS
RPW = B // NW
IPW = RPW * F
CHUNK = 128
CPF = RPW // CHUNK
NCHUNK = IPW // CHUNK


def _body(xtw_hbm, wflat_hbm, bias_hbm, out_hbm, xbuf, gbuf, bias_v, obuf, sem):
    wid = lax.axis_index("s") * NC + lax.axis_index("c")

    pltpu.sync_copy(xtw_hbm.at[wid], xbuf)
    pltpu.sync_copy(bias_hbm, bias_v)

    def fire(k, _):
        f = k // CPF
        off = (f * FIELD_DIM).astype(jnp.int32)
        row = f
        cb = (k % CPF) * CHUNK
        for c in range(CHUNK // L):
            sl = pl.ds(cb + c * L, L)
            xbuf[row, sl] = xbuf[row, sl] + off
        pltpu.async_copy(
            wflat_hbm.at[xbuf.at[row, pl.ds(cb, CHUNK)]],
            gbuf.at[pl.ds(k * CHUNK, CHUNK)],
            sem,
        )
        return 0

    lax.fori_loop(0, NCHUNK, fire, 0)

    pltpu.make_async_copy(wflat_hbm.at[pl.ds(0, IPW)], gbuf, sem).wait()

    bias_vec = bias_v[...]

    def reduce(j, _):
        base = j * L
        vacc = bias_vec
        for f in range(F):
            vacc = vacc + gbuf[pl.ds(f * RPW + base, L)]
        obuf[pl.ds(base, L)] = 1.0 / (1.0 + jnp.exp(-vacc))
        return 0

    lax.fori_loop(0, RPW // L, reduce, 0)

    pltpu.sync_copy(obuf, out_hbm.at[pl.ds(wid * RPW, RPW)])


@jax.jit
def kernel(x, weight, bias):
    eye = jnp.broadcast_to(jnp.eye(F, dtype=jnp.float32), (NW, F, F))
    xt = lax.dot_general(
        eye, x.astype(jnp.float32).reshape(NW, RPW, F),
        dimension_numbers=(((2,), (2,)), ((0,), (0,))),
        preferred_element_type=jnp.float32,
        precision=lax.Precision.HIGHEST,
    )
    xtw = xt.astype(jnp.int32)
    wflat = weight.reshape(-1)
    bias16 = jnp.broadcast_to(bias, (L,))

    mesh = plsc.VectorSubcoreMesh(core_axis_name="c", subcore_axis_name="s")
    run = pl.kernel(
        _body,
        out_type=jax.ShapeDtypeStruct((B,), jnp.float32),
        mesh=mesh,
        scratch_types=[
            pltpu.VMEM((F, RPW), jnp.int32),
            pltpu.VMEM((IPW,), jnp.float32),
            pltpu.VMEM((L,), jnp.float32),
            pltpu.VMEM((RPW,), jnp.float32),
            pltpu.SemaphoreType.DMA,
        ],
    )
    return run(xtw, wflat, bias16)


# speedup vs baseline: 1.9366x; 1.9366x over previous
import jax
import jax.numpy as jnp
from jax import lax
from jax.experimental import pallas as pl
from jax.experimental.pallas import tpu as pltpu
from jax.experimental.pallas import tpu_sc as plsc

B = 16384
F = 26
FIELD_DIM = 38461
NC, NS, L = 2, 16, 16
NW = NC * NS
RPW = B // NW
IPW = RPW * F
CHUNK = 128
CPF = RPW // CHUNK
NCHUNK = IPW // CHUNK


def _body(xtw_hbm, wrow_hbm, bias_hbm, out_hbm, xbuf, gbuf, bias_v, obuf, sem):
    wid = lax.axis_index("s") * NC + lax.axis_index("c")

    pltpu.sync_copy(xtw_hbm.at[wid], xbuf)
    pltpu.sync_copy(bias_hbm, bias_v)

    def fire(k, _):
        row = k // CPF
        cb = (k % CPF) * CHUNK
        pltpu.async_copy(
            wrow_hbm.at[xbuf.at[pl.ds(row, 1), pl.ds(cb, CHUNK)]],
            gbuf.at[pl.ds(0, 1), pl.ds(k * CHUNK, CHUNK)],
            sem,
        )
        return 0

    lax.fori_loop(0, NCHUNK, fire, 0)

    pltpu.make_async_copy(
        wrow_hbm.at[pl.ds(0, 1), pl.ds(0, IPW)], gbuf, sem
    ).wait()

    bias_vec = bias_v[...]
    gflat = gbuf.at[0]

    def reduce(j, _):
        base = j * L
        vacc = bias_vec
        for f in range(F):
            vacc = vacc + gflat[pl.ds(f * RPW + base, L)]
        obuf[pl.ds(base, L)] = 1.0 / (1.0 + jnp.exp(-vacc))
        return 0

    lax.fori_loop(0, RPW // L, reduce, 0)

    pltpu.sync_copy(obuf, out_hbm.at[pl.ds(wid * RPW, RPW)])


@jax.jit
def kernel(x, weight, bias):
    offs = jnp.arange(F, dtype=jnp.float32) * FIELD_DIM
    lhs = jnp.concatenate([jnp.eye(F, dtype=jnp.float32), offs[:, None]], axis=1)
    lhs = jnp.broadcast_to(lhs, (NW, F, F + 1))
    xf = x.astype(jnp.float32).reshape(NW, RPW, F)
    ones = jnp.ones((NW, RPW, 1), dtype=jnp.float32)
    rhs = jnp.concatenate([xf, ones], axis=2)
    xt = lax.dot_general(
        lhs, rhs,
        dimension_numbers=(((2,), (2,)), ((0,), (0,))),
        preferred_element_type=jnp.float32,
        precision=lax.Precision.HIGHEST,
    )
    xtw = xt.astype(jnp.int32)
    bias16 = jnp.broadcast_to(bias, (L,))

    mesh = plsc.VectorSubcoreMesh(core_axis_name="c", subcore_axis_name="s")
    run = pl.kernel(
        _body,
        out_type=jax.ShapeDtypeStruct((B,), jnp.float32),
        mesh=mesh,
        scratch_types=[
            pltpu.VMEM((F, RPW), jnp.int32),
            pltpu.VMEM((1, IPW), jnp.float32),
            pltpu.VMEM((L,), jnp.float32),
            pltpu.VMEM((RPW,), jnp.float32),
            pltpu.SemaphoreType.DMA,
        ],
    )
    return run(xtw, weight.reshape(1, -1), bias16)


# speedup vs baseline: 1.9681x; 1.0163x over previous
import jax
import jax.numpy as jnp
from jax import lax
from jax.experimental import pallas as pl
from jax.experimental.pallas import tpu as pltpu
from jax.experimental.pallas import tpu_sc as plsc

B = 16384
F = 26
FIELD_DIM = 38461
NC, NS, L = 2, 16, 16
NW = NC * NS
RPW = B // NW
IPW = RPW * F
CHUNK = 128
NRG = RPW // CHUNK
NCHUNK = IPW // CHUNK


def _body(xtw_hbm, wrow_hbm, bias_hbm, out_hbm, xbuf, gbuf, bias_v, obuf,
          sem0, sem1, sem2, sem3):
    wid = lax.axis_index("s") * NC + lax.axis_index("c")
    sems = [sem0, sem1, sem2, sem3]

    pltpu.sync_copy(bias_hbm, bias_v)

    for rg in range(NRG):
        pltpu.sync_copy(xtw_hbm.at[wid, rg], xbuf.at[rg])

        def fire(f, _, rg=rg):
            pltpu.async_copy(
                wrow_hbm.at[xbuf.at[rg, pl.ds(f, 1)]],
                gbuf.at[pl.ds(0, 1), pl.ds((rg * F + f) * CHUNK, CHUNK)],
                sem=sems[rg],
            )
            return 0

        lax.fori_loop(0, F, fire, 0)

    bias_vec = bias_v[...]
    gflat = gbuf.at[0]
    for rg in range(NRG):
        pltpu.make_async_copy(
            wrow_hbm.at[pl.ds(0, 1), pl.ds(0, F * CHUNK)],
            gbuf.at[pl.ds(0, 1), pl.ds(rg * F * CHUNK, F * CHUNK)],
            sems[rg],
        ).wait()

        def reduce(j, _, rg=rg):
            base = rg * F * CHUNK + j * L
            vacc = bias_vec
            for f in range(F):
                vacc = vacc + gflat[pl.ds(base + f * CHUNK, L)]
            obuf[pl.ds(rg * CHUNK + j * L, L)] = 1.0 / (1.0 + jnp.exp(-vacc))
            return 0

        lax.fori_loop(0, CHUNK // L, reduce, 0)

        pltpu.sync_copy(
            obuf.at[pl.ds(rg * CHUNK, CHUNK)],
            out_hbm.at[pl.ds(wid * RPW + rg * CHUNK, CHUNK)],
        )


@jax.jit
def kernel(x, weight, bias):
    offs = jnp.arange(F, dtype=jnp.float32) * FIELD_DIM
    lhs = jnp.concatenate([jnp.eye(F, dtype=jnp.float32), offs[:, None]], axis=1)
    lhs = jnp.broadcast_to(lhs, (NW, NRG, F, F + 1))
    xf = x.astype(jnp.float32).reshape(NW, NRG, CHUNK, F)
    ones = jnp.ones((NW, NRG, CHUNK, 1), dtype=jnp.float32)
    rhs = jnp.concatenate([xf, ones], axis=3)
    xt = lax.dot_general(
        lhs, rhs,
        dimension_numbers=(((3,), (3,)), ((0, 1), (0, 1))),
        preferred_element_type=jnp.float32,
        precision=lax.Precision.HIGHEST,
    )
    xtw = xt.astype(jnp.int32)
    bias16 = jnp.broadcast_to(bias, (L,))

    mesh = plsc.VectorSubcoreMesh(core_axis_name="c", subcore_axis_name="s")
    run = pl.kernel(
        _body,
        out_type=jax.ShapeDtypeStruct((B,), jnp.float32),
        mesh=mesh,
        scratch_types=[
            pltpu.VMEM((NRG, F, CHUNK), jnp.int32),
            pltpu.VMEM((1, IPW), jnp.float32),
            pltpu.VMEM((L,), jnp.float32),
            pltpu.VMEM((RPW,), jnp.float32),
            pltpu.SemaphoreType.DMA,
            pltpu.SemaphoreType.DMA,
            pltpu.SemaphoreType.DMA,
            pltpu.SemaphoreType.DMA,
        ],
    )
    return run(xtw, weight.reshape(1, -1), bias16)


# speedup vs baseline: 1.9885x; 1.0104x over previous
import jax
import jax.numpy as jnp
from jax import lax
from jax.experimental import pallas as pl
from jax.experimental.pallas import tpu as pltpu
from jax.experimental.pallas import tpu_sc as plsc

B = 16384
F = 26
FIELD_DIM = 38461
NC, NS, L = 2, 16, 16
NW = NC * NS
RPW = B // NW
IPW = RPW * F
CHUNK = 128
NRG = RPW // CHUNK
NCHUNK = IPW // CHUNK


def _body(xtw_hbm, wrow_hbm, bias_hbm, out_hbm, xbuf, gbuf, bias_v, obuf,
          sem0, sem1, sem2, sem3):
    wid = lax.axis_index("s") * NC + lax.axis_index("c")
    sems = [sem0, sem1, sem2, sem3]

    pltpu.sync_copy(bias_hbm, bias_v)

    for rg in range(NRG):
        pltpu.sync_copy(xtw_hbm.at[wid, rg], xbuf.at[rg])

        def fire(f, _, rg=rg):
            pltpu.async_copy(
                wrow_hbm.at[xbuf.at[rg, pl.ds(f, 1)]],
                gbuf.at[pl.ds(0, 1), pl.ds((rg * F + f) * CHUNK, CHUNK)],
                sem=sems[rg],
            )
            return 0

        lax.fori_loop(0, F, fire, 0)

    bias_vec = bias_v[...]
    gflat = gbuf.at[0]
    for rg in range(NRG):
        pltpu.make_async_copy(
            wrow_hbm.at[pl.ds(0, 1), pl.ds(0, F * CHUNK)],
            gbuf.at[pl.ds(0, 1), pl.ds(rg * F * CHUNK, F * CHUNK)],
            sems[rg],
        ).wait()

        def reduce(j, _, rg=rg):
            base = rg * F * CHUNK + j * L
            vacc = bias_vec
            for f in range(F):
                vacc = vacc + gflat[pl.ds(base + f * CHUNK, L)]
            obuf[pl.ds(rg * CHUNK + j * L, L)] = 1.0 / (1.0 + jnp.exp(-vacc))
            return 0

        lax.fori_loop(0, CHUNK // L, reduce, 0)

        pltpu.sync_copy(
            obuf.at[pl.ds(rg * CHUNK, CHUNK)],
            out_hbm.at[pl.ds(wid * RPW + rg * CHUNK, CHUNK)],
        )


@jax.jit
def kernel(x, weight, bias):
    offs = jnp.arange(F, dtype=jnp.float32) * FIELD_DIM
    lhs = jnp.broadcast_to(jnp.eye(F, dtype=jnp.float32), (NW, NRG, F, F))
    rhs = x.astype(jnp.float32).reshape(NW, NRG, CHUNK, F)
    xt = lax.dot_general(
        lhs, rhs,
        dimension_numbers=(((3,), (3,)), ((0, 1), (0, 1))),
        preferred_element_type=jnp.float32,
        precision=lax.Precision.HIGHEST,
    ) + offs[None, None, :, None]
    xtw = xt.astype(jnp.int32)
    bias16 = jnp.broadcast_to(bias, (L,))

    mesh = plsc.VectorSubcoreMesh(core_axis_name="c", subcore_axis_name="s")
    run = pl.kernel(
        _body,
        out_type=jax.ShapeDtypeStruct((B,), jnp.float32),
        mesh=mesh,
        scratch_types=[
            pltpu.VMEM((NRG, F, CHUNK), jnp.int32),
            pltpu.VMEM((1, IPW), jnp.float32),
            pltpu.VMEM((L,), jnp.float32),
            pltpu.VMEM((RPW,), jnp.float32),
            pltpu.SemaphoreType.DMA,
            pltpu.SemaphoreType.DMA,
            pltpu.SemaphoreType.DMA,
            pltpu.SemaphoreType.DMA,
        ],
    )
    return run(xtw, weight.reshape(1, -1), bias16)


# speedup vs baseline: 1.9936x; 1.0026x over previous
import jax
import jax.numpy as jnp
from jax import lax
from jax.experimental import pallas as pl
from jax.experimental.pallas import tpu as pltpu
from jax.experimental.pallas import tpu_sc as plsc

B = 16384
F = 26
FIELD_DIM = 38461
NC, NS, L = 2, 16, 16
NW = NC * NS
RPW = B // NW
IPW = RPW * F
CHUNK = 128
NRG = RPW // CHUNK
NCHUNK = IPW // CHUNK


def _body(xtw_hbm, wrow_hbm, bias_hbm, out_hbm, xbuf, gbuf, bias_v, obuf,
          sem0, sem1, sem2, sem3):
    wid = lax.axis_index("s") * NC + lax.axis_index("c")
    sems = [sem0, sem1, sem2, sem3]

    pltpu.sync_copy(bias_hbm, bias_v)

    stages = [
        pltpu.make_async_copy(xtw_hbm.at[wid, rg], xbuf.at[rg], sems[rg])
        for rg in range(NRG)
    ]
    for st in stages:
        st.start()

    for rg in range(NRG):
        stages[rg].wait()

        def fire(f, _, rg=rg):
            for i in range(2):
                pltpu.async_copy(
                    wrow_hbm.at[xbuf.at[rg, pl.ds(f * 2 + i, 1)]],
                    gbuf.at[
                        pl.ds(0, 1),
                        pl.ds((rg * F + f * 2 + i) * CHUNK, CHUNK),
                    ],
                    sem=sems[rg],
                )
            return 0

        lax.fori_loop(0, F // 2, fire, 0)

    bias_vec = bias_v[...]
    gflat = gbuf.at[0]
    for rg in range(NRG):
        pltpu.make_async_copy(
            wrow_hbm.at[pl.ds(0, 1), pl.ds(0, F * CHUNK)],
            gbuf.at[pl.ds(0, 1), pl.ds(rg * F * CHUNK, F * CHUNK)],
            sems[rg],
        ).wait()

        def reduce(j, _, rg=rg):
            base = rg * F * CHUNK + j * L
            vacc = bias_vec
            for f in range(F):
                vacc = vacc + gflat[pl.ds(base + f * CHUNK, L)]
            obuf[pl.ds(rg * CHUNK + j * L, L)] = 1.0 / (1.0 + jnp.exp(-vacc))
            return 0

        lax.fori_loop(0, CHUNK // L, reduce, 0)

        pltpu.sync_copy(
            obuf.at[pl.ds(rg * CHUNK, CHUNK)],
            out_hbm.at[pl.ds(wid * RPW + rg * CHUNK, CHUNK)],
        )


@jax.jit
def kernel(x, weight, bias):
    offs = jnp.arange(F, dtype=jnp.float32) * FIELD_DIM
    lhs = jnp.broadcast_to(jnp.eye(F, dtype=jnp.float32), (NW, NRG, F, F))
    rhs = x.astype(jnp.float32).reshape(NW, NRG, CHUNK, F)
    xt = lax.dot_general(
        lhs, rhs,
        dimension_numbers=(((3,), (3,)), ((0, 1), (0, 1))),
        preferred_element_type=jnp.float32,
        precision=lax.Precision.HIGHEST,
    ) + offs[None, None, :, None]
    xtw = xt.astype(jnp.int32)
    bias16 = jnp.broadcast_to(bias, (L,))

    mesh = plsc.VectorSubcoreMesh(core_axis_name="c", subcore_axis_name="s")
    run = pl.kernel(
        _body,
        out_type=jax.ShapeDtypeStruct((B,), jnp.float32),
        mesh=mesh,
        scratch_types=[
            pltpu.VMEM((NRG, F, CHUNK), jnp.int32),
            pltpu.VMEM((1, IPW), jnp.float32),
            pltpu.VMEM((L,), jnp.float32),
            pltpu.VMEM((RPW,), jnp.float32),
            pltpu.SemaphoreType.DMA,
            pltpu.SemaphoreType.DMA,
            pltpu.SemaphoreType.DMA,
            pltpu.SemaphoreType.DMA,
        ],
    )
    return run(xtw, weight.reshape(1, -1), bias16)


# speedup vs baseline: 1.9968x; 1.0016x over previous
import jax
import jax.numpy as jnp
from jax import lax
from jax.experimental import pallas as pl
from jax.experimental.pallas import tpu as pltpu
from jax.experimental.pallas import tpu_sc as plsc

B = 16384
F = 26
FIELD_DIM = 38461
NC, NS, L = 2, 16, 16
NW = NC * NS
RPW = B // NW
IPW = RPW * F
CHUNK = 128
NRG = RPW // CHUNK
NCHUNK = IPW // CHUNK


def _body(xtw_hbm, wrow_hbm, bias_hbm, out_hbm, xbuf, gbuf, bias_v, obuf,
          sem0, sem1, sem2, sem3):
    wid = lax.axis_index("s") * NC + lax.axis_index("c")
    sems = [sem0, sem1, sem2, sem3]

    pltpu.sync_copy(bias_hbm, bias_v)

    stages = [
        pltpu.make_async_copy(xtw_hbm.at[wid, rg], xbuf.at[rg], sems[rg])
        for rg in range(NRG)
    ]
    for st in stages:
        st.start()

    for rg in range(NRG):
        stages[rg].wait()

        def fire(f, _, rg=rg):
            for i in range(2):
                pltpu.async_copy(
                    wrow_hbm.at[xbuf.at[rg, pl.ds(f * 2 + i, 1)]],
                    gbuf.at[
                        pl.ds(0, 1),
                        pl.ds((rg * F + f * 2 + i) * CHUNK, CHUNK),
                    ],
                    sem=sems[rg],
                )
            return 0

        lax.fori_loop(0, F // 2, fire, 0)

    bias_vec = bias_v[...]
    gflat = gbuf.at[0]
    for rg in range(NRG):
        pltpu.make_async_copy(
            wrow_hbm.at[pl.ds(0, 1), pl.ds(0, F * CHUNK)],
            gbuf.at[pl.ds(0, 1), pl.ds(rg * F * CHUNK, F * CHUNK)],
            sems[rg],
        ).wait()

        def reduce(j, _, rg=rg):
            base = rg * F * CHUNK + j * L
            vacc = bias_vec
            for f in range(F):
                vacc = vacc + gflat[pl.ds(base + f * CHUNK, L)]
            obuf[pl.ds(rg * CHUNK + j * L, L)] = 1.0 / (1.0 + jnp.exp(-vacc))
            return 0

        lax.fori_loop(0, CHUNK // L, reduce, 0)

        pltpu.sync_copy(
            obuf.at[pl.ds(rg * CHUNK, CHUNK)],
            out_hbm.at[pl.ds(wid * RPW + rg * CHUNK, CHUNK)],
        )


@jax.jit
def kernel(x, weight, bias):
    offs = jnp.arange(F, dtype=jnp.float32) * FIELD_DIM
    lhs = jnp.broadcast_to(jnp.eye(F, dtype=jnp.float32), (NW, NRG, F, F))
    rhs = x.reshape(NW, NRG, CHUNK, F).astype(jnp.float32)
    xt = lax.dot_general(
        lhs, rhs,
        dimension_numbers=(((3,), (3,)), ((0, 1), (0, 1))),
        preferred_element_type=jnp.float32,
        precision=lax.Precision.HIGHEST,
    ) + offs[None, None, :, None]
    xtw = xt.astype(jnp.int32)
    bias16 = jnp.broadcast_to(bias, (L,))

    mesh = plsc.VectorSubcoreMesh(core_axis_name="c", subcore_axis_name="s")
    run = pl.kernel(
        _body,
        out_type=jax.ShapeDtypeStruct((B,), jnp.float32),
        mesh=mesh,
        scratch_types=[
            pltpu.VMEM((NRG, F, CHUNK), jnp.int32),
            pltpu.VMEM((1, IPW), jnp.float32),
            pltpu.VMEM((L,), jnp.float32),
            pltpu.VMEM((RPW,), jnp.float32),
            pltpu.SemaphoreType.DMA,
            pltpu.SemaphoreType.DMA,
            pltpu.SemaphoreType.DMA,
            pltpu.SemaphoreType.DMA,
        ],
    )
    return run(xtw, weight.reshape(1, -1), bias16)
